# baseline (speedup 1.0000x reference)
import functools

import jax
import jax.numpy as jnp
from jax import lax
from jax.experimental import pallas as pl
from jax.experimental.pallas import tpu as pltpu
from jax.experimental.pallas import tpu_sc as plsc

N = 10000
E = 320000
F = 128
C = 128
NUM_ITER = 2

NS = 16
CH = 64
IB = 16
EDGES = 2 * E
HALF = N // 2
K2 = 320
KB = K2 // IB
CAP = NS * K2 * CH
N_LOC = 5120
N_ACC = 2 * N_LOC
TILE_ROWS = N_ACC // NS
TRASH = N_LOC - 2

BLK = 2000
GRID = N // BLK


def _init_body(obj_ref, wos_ref, wsa_ref, wlsa_ref, bs_ref, ba_ref,
               st_ref, m_ref, ea_ref):
    x = obj_ref[...]
    st = jnp.tanh(jnp.dot(x, wos_ref[...], preferred_element_type=jnp.float32)
                  + bs_ref[...])
    a = jnp.dot(st, wsa_ref[...], preferred_element_type=jnp.float32)
    l = jnp.dot(st, wlsa_ref[...], preferred_element_type=jnp.float32)
    p = jnp.exp(l)
    st_ref[...] = st
    m_ref[...] = jnp.concatenate([p, p * st], axis=1)
    ea_ref[...] = jnp.exp(a + ba_ref[...])


def _mid_body(st_ref, s_ref, t_ref, ea_ref, wls_ref, wsa_ref, wlsa_ref,
              bs_ref, ba_ref, nst_ref, m_ref, nea_ref):
    st = st_ref[...]
    ea = ea_ref[...]
    inv = 1.0 / (ea * s_ref[...] + 1.0)
    g = ea * inv * t_ref[...]
    nst = jnp.tanh(st * inv
                   + jnp.dot(g, wls_ref[...], preferred_element_type=jnp.float32)
                   + bs_ref[...])
    a = jnp.dot(nst, wsa_ref[...], preferred_element_type=jnp.float32)
    l = jnp.dot(nst, wlsa_ref[...], preferred_element_type=jnp.float32)
    p = jnp.exp(l)
    nst_ref[...] = nst
    m_ref[...] = jnp.concatenate([p, p * nst], axis=1)
    nea_ref[...] = jnp.exp(a + ba_ref[...])


def _final_body(st_ref, s_ref, t_ref, ea_ref, wls_ref, bs_ref, out_ref):
    st = st_ref[...]
    ea = ea_ref[...]
    inv = 1.0 / (ea * s_ref[...] + 1.0)
    g = ea * inv * t_ref[...]
    out_ref[...] = jnp.tanh(
        st * inv
        + jnp.dot(g, wls_ref[...], preferred_element_type=jnp.float32)
        + bs_ref[...])


_row_spec = pl.BlockSpec((BLK, C), lambda i: (i, 0))
_wide_spec = pl.BlockSpec((BLK, 2 * C), lambda i: (i, 0))
_w_spec = pl.BlockSpec((C, C), lambda i: (0, 0))
_b_spec = pl.BlockSpec((1, C), lambda i: (0, 0))
_nc_shape = jax.ShapeDtypeStruct((N, C), jnp.float32)
_wide_shape = jax.ShapeDtypeStruct((N, 2 * C), jnp.float32)


def _tc_init(obj, wos, wsa, wlsa, bs2, ba2):
    return pl.pallas_call(
        _init_body,
        grid=(GRID,),
        in_specs=[_row_spec, _w_spec, _w_spec, _w_spec, _b_spec, _b_spec],
        out_specs=[_row_spec, _wide_spec, _row_spec],
        out_shape=[_nc_shape, _wide_shape, _nc_shape],
    )(obj, wos, wsa, wlsa, bs2, ba2)


def _tc_mid(st, s, t, ea, wls, wsa, wlsa, bs2, ba2):
    return pl.pallas_call(
        _mid_body,
        grid=(GRID,),
        in_specs=[_row_spec] * 4 + [_w_spec] * 3 + [_b_spec] * 2,
        out_specs=[_row_spec, _wide_spec, _row_spec],
        out_shape=[_nc_shape, _wide_shape, _nc_shape],
    )(st, s, t, ea, wls, wsa, wlsa, bs2, ba2)


def _tc_final(st, s, t, ea, wls, bs2):
    return pl.pallas_call(
        _final_body,
        grid=(GRID,),
        in_specs=[_row_spec] * 4 + [_w_spec, _b_spec],
        out_specs=_row_spec,
        out_shape=_nc_shape,
    )(st, s, t, ea, wls, bs2)


@functools.lru_cache(maxsize=1)
def _build_segsum():
    @functools.partial(
        pl.kernel,
        out_type=jax.ShapeDtypeStruct((2, N_ACC, C), jnp.float32),
        mesh=plsc.VectorSubcoreMesh(core_axis_name="c", subcore_axis_name="s",
                                    num_cores=2, num_subcores=NS),
        scratch_types=[
            pltpu.VMEM((IB, 2 * CH), jnp.int32),
            pltpu.VMEM((IB, 2 * CH), jnp.int32),
            pltpu.VMEM((2 * CH, C), jnp.float32),
            pltpu.VMEM((2 * CH, C), jnp.float32),
            pltpu.VMEM_SHARED((N_ACC, C), jnp.float32),
            pltpu.SemaphoreType.DMA,
        ],
    )
    def _segsum(m_hbm, z_hbm, ct_hbm, cf_hbm, out,
                ct_v, cf_v, rows0, rows1, acc, sem):
        cid = lax.axis_index("c")
        sid = lax.axis_index("s")
        r0 = sid * TILE_ROWS
        pltpu.sync_copy(z_hbm.at[pl.ds(r0, TILE_ROWS)],
                        acc.at[pl.ds(r0, TILE_ROWS)])
        plsc.subcore_barrier()

        bufs = (rows0, rows1)

        w = cid * NS + sid

        def block(j, carry):
            pltpu.sync_copy(ct_hbm.at[w, pl.ds(j * IB, IB)], ct_v)
            pltpu.sync_copy(cf_hbm.at[w, pl.ds(j * IB, IB)], cf_v)
            desc = pltpu.async_copy(m_hbm.at[ct_v.at[0]], bufs[0], sem)
            for k in range(IB):
                desc.wait()
                if k + 1 < IB:
                    desc = pltpu.async_copy(m_hbm.at[ct_v.at[k + 1]],
                                            bufs[(k + 1) % 2], sem)
                pltpu.sync_copy(bufs[k % 2], acc.at[cf_v.at[k]], add=True)
            return carry

        lax.fori_loop(0, KB, block, 0)
        plsc.subcore_barrier()
        pltpu.sync_copy(acc.at[pl.ds(r0, TILE_ROWS)],
                        out.at[cid, pl.ds(r0, TILE_ROWS)])

    return _segsum


def kernel(objects, connections, object_state_W, state_attention_W,
           linked_state_attention_W, attention_b, linked_state_W, state_b):
    obj = objects[0]
    u = connections[0, :, 0]
    v = connections[0, :, 1]
    src = jnp.concatenate([v, u])
    dst = jnp.concatenate([u, v])
    h = (dst >= HALF).astype(jnp.int32)
    c1 = jnp.cumsum(h)
    i_arr = jnp.arange(EDGES, dtype=jnp.int32)
    slot = jnp.where(h == 1, CAP + c1 - 1, i_arr - c1)
    gidx = jnp.zeros((2 * CAP,), jnp.int32).at[slot].set(src)
    lrow = dst - h * HALF
    sidx = jnp.full((2 * CAP,), TRASH, jnp.int32).at[slot].set(lrow)
    gp = jnp.stack([2 * gidx, 2 * gidx + 1], axis=-1)
    pidx = jnp.stack([2 * sidx, 2 * sidx + 1], axis=-1)
    ct_idx = gp.reshape(2 * NS, K2, 2 * CH)
    cf_idx = pidx.reshape(2 * NS, K2, 2 * CH)
    zeros = jnp.zeros((N_ACC, C), jnp.float32)

    bs2 = state_b.reshape(1, C)
    ba2 = attention_b.reshape(1, C)

    st, m, ea = _tc_init(obj, object_state_W, state_attention_W,
                         linked_state_attention_W, bs2, ba2)
    for it in range(NUM_ITER):
        o = _build_segsum()(m.reshape(2 * N, C), zeros, ct_idx, cf_idx)
        o0 = o[0].reshape(N_LOC, 2, C)
        o1 = o[1].reshape(N_LOC, 2, C)
        s = jnp.concatenate([o0[:HALF, 0], o1[:HALF, 0]], axis=0)
        t = jnp.concatenate([o0[:HALF, 1], o1[:HALF, 1]], axis=0)
        if it < NUM_ITER - 1:
            st, m, ea = _tc_mid(st, s, t, ea, linked_state_W,
                                state_attention_W, linked_state_attention_W,
                                bs2, ba2)
        else:
            st = _tc_final(st, s, t, ea, linked_state_W, bs2)
    return st[None]


# speedup vs baseline: 2.8299x; 2.8299x over previous
import functools

import jax
import jax.numpy as jnp
from jax import lax
from jax.experimental import pallas as pl
from jax.experimental.pallas import tpu as pltpu
from jax.experimental.pallas import tpu_sc as plsc

N = 10000
E = 320000
F = 128
C = 128
NUM_ITER = 2

NS = 16
CHUNK = 128
IB = 16
EDGES = 2 * E
K = IB * (-(-EDGES // (NS * CHUNK * IB)))
KB = K // IB
EDGES_PAD = NS * K * CHUNK
ROWS_PER_TILE = 640
N_ACC = NS * ROWS_PER_TILE
TRASH_ROW = N_ACC - 2

BLK = 2000
GRID = N // BLK


def _init_body(obj_ref, wos_ref, wsa_ref, wlsa_ref, bs_ref, ba_ref,
               st_ref, p_ref, ps_ref, ea_ref):
    x = obj_ref[...]
    st = jnp.tanh(jnp.dot(x, wos_ref[...], preferred_element_type=jnp.float32)
                  + bs_ref[...])
    a = jnp.dot(st, wsa_ref[...], preferred_element_type=jnp.float32)
    l = jnp.dot(st, wlsa_ref[...], preferred_element_type=jnp.float32)
    p = jnp.exp(l)
    st_ref[...] = st
    p_ref[...] = p
    ps_ref[...] = p * st
    ea_ref[...] = jnp.exp(a + ba_ref[...])


def _mid_body(st_ref, s_ref, t_ref, ea_ref, wls_ref, wsa_ref, wlsa_ref,
              bs_ref, ba_ref, nst_ref, p_ref, ps_ref, nea_ref):
    st = st_ref[...]
    ea = ea_ref[...]
    inv = 1.0 / (ea * s_ref[...] + 1.0)
    g = ea * inv * t_ref[...]
    nst = jnp.tanh(st * inv
                   + jnp.dot(g, wls_ref[...], preferred_element_type=jnp.float32)
                   + bs_ref[...])
    a = jnp.dot(nst, wsa_ref[...], preferred_element_type=jnp.float32)
    l = jnp.dot(nst, wlsa_ref[...], preferred_element_type=jnp.float32)
    p = jnp.exp(l)
    nst_ref[...] = nst
    p_ref[...] = p
    ps_ref[...] = p * nst
    nea_ref[...] = jnp.exp(a + ba_ref[...])


def _final_body(st_ref, s_ref, t_ref, ea_ref, wls_ref, bs_ref, out_ref):
    st = st_ref[...]
    ea = ea_ref[...]
    inv = 1.0 / (ea * s_ref[...] + 1.0)
    g = ea * inv * t_ref[...]
    out_ref[...] = jnp.tanh(
        st * inv
        + jnp.dot(g, wls_ref[...], preferred_element_type=jnp.float32)
        + bs_ref[...])


_row_spec = pl.BlockSpec((BLK, C), lambda i: (i, 0))
_w_spec = pl.BlockSpec((C, C), lambda i: (0, 0))
_b_spec = pl.BlockSpec((1, C), lambda i: (0, 0))
_nc_shape = jax.ShapeDtypeStruct((N, C), jnp.float32)


def _tc_init(obj, wos, wsa, wlsa, bs2, ba2):
    return pl.pallas_call(
        _init_body,
        grid=(GRID,),
        in_specs=[_row_spec, _w_spec, _w_spec, _w_spec, _b_spec, _b_spec],
        out_specs=[_row_spec] * 4,
        out_shape=[_nc_shape] * 4,
    )(obj, wos, wsa, wlsa, bs2, ba2)


def _tc_mid(st, s, t, ea, wls, wsa, wlsa, bs2, ba2):
    return pl.pallas_call(
        _mid_body,
        grid=(GRID,),
        in_specs=[_row_spec] * 4 + [_w_spec] * 3 + [_b_spec] * 2,
        out_specs=[_row_spec] * 4,
        out_shape=[_nc_shape] * 4,
    )(st, s, t, ea, wls, wsa, wlsa, bs2, ba2)


def _tc_final(st, s, t, ea, wls, bs2):
    return pl.pallas_call(
        _final_body,
        grid=(GRID,),
        in_specs=[_row_spec] * 4 + [_w_spec, _b_spec],
        out_specs=_row_spec,
        out_shape=_nc_shape,
    )(st, s, t, ea, wls, bs2)


@functools.lru_cache(maxsize=1)
def _build_segsum():
    @functools.partial(
        pl.kernel,
        out_type=jax.ShapeDtypeStruct((2, N_ACC, C), jnp.float32),
        mesh=plsc.VectorSubcoreMesh(core_axis_name="c", subcore_axis_name="s",
                                    num_cores=2, num_subcores=NS),
        scratch_types=[
            pltpu.VMEM((IB, CHUNK), jnp.int32),
            pltpu.VMEM((IB, CHUNK), jnp.int32),
            pltpu.VMEM((CHUNK, C), jnp.float32),
            pltpu.VMEM((CHUNK, C), jnp.float32),
            pltpu.VMEM_SHARED((N_ACC, C), jnp.float32),
            pltpu.SemaphoreType.DMA,
        ],
    )
    def _segsum(p_hbm, ps_hbm, z_hbm, ct_hbm, cf_hbm, out,
                ct_v, cf_v, rows0, rows1, acc, sem):
        cid = lax.axis_index("c")
        sid = lax.axis_index("s")
        r0 = sid * ROWS_PER_TILE
        pltpu.sync_copy(z_hbm.at[pl.ds(r0, ROWS_PER_TILE)],
                        acc.at[pl.ds(r0, ROWS_PER_TILE)])
        plsc.subcore_barrier()

        bufs = (rows0, rows1)
        H = CHUNK // 2

        def gather(tbl, k, b):
            d0 = pltpu.async_copy(tbl.at[ct_v.at[k, pl.ds(0, H)]],
                                  bufs[b].at[pl.ds(0, H)], sem)
            d1 = pltpu.async_copy(tbl.at[ct_v.at[k, pl.ds(H, H)]],
                                  bufs[b].at[pl.ds(H, H)], sem)
            return (d0, d1)

        def run(tbl):
            def block(j, carry):
                pltpu.sync_copy(ct_hbm.at[sid, pl.ds(j * IB, IB)], ct_v)
                pltpu.sync_copy(cf_hbm.at[sid, pl.ds(j * IB, IB)], cf_v)
                descs = gather(tbl, 0, 0)
                for k in range(IB):
                    for d in descs:
                        d.wait()
                    if k + 1 < IB:
                        descs = gather(tbl, k + 1, (k + 1) % 2)
                    pltpu.sync_copy(bufs[k % 2], acc.at[cf_v.at[k]], add=True)
                return carry
            lax.fori_loop(0, KB, block, 0)
            plsc.subcore_barrier()
            pltpu.sync_copy(acc.at[pl.ds(r0, ROWS_PER_TILE)],
                            out.at[cid, pl.ds(r0, ROWS_PER_TILE)])

        @pl.when(cid == 0)
        def _():
            run(p_hbm)

        @pl.when(cid == 1)
        def _():
            run(ps_hbm)

    return _segsum


def kernel(objects, connections, object_state_W, state_attention_W,
           linked_state_attention_W, attention_b, linked_state_W, state_b):
    obj = objects[0]
    u = connections[0, :, 0]
    v = connections[0, :, 1]
    gat = jnp.concatenate([v, u])
    sca = jnp.concatenate([u, v])
    pad = EDGES_PAD - EDGES
    gat = jnp.concatenate([gat, jnp.zeros((pad,), jnp.int32)])
    sca = jnp.concatenate([sca, jnp.full((pad,), TRASH_ROW, jnp.int32)])
    ct_idx = gat.reshape(NS, K, CHUNK)
    cf_idx = sca.reshape(NS, K, CHUNK)
    zeros = jnp.zeros((N_ACC, C), jnp.float32)

    bs2 = state_b.reshape(1, C)
    ba2 = attention_b.reshape(1, C)

    st, p, ps, ea = _tc_init(obj, object_state_W, state_attention_W,
                             linked_state_attention_W, bs2, ba2)
    for it in range(NUM_ITER):
        o = _build_segsum()(p, ps, zeros, ct_idx, cf_idx)
        s_pad = o[0]
        t_pad = o[1]
        if it < NUM_ITER - 1:
            st, p, ps, ea = _tc_mid(st, s_pad, t_pad, ea, linked_state_W,
                                    state_attention_W, linked_state_attention_W,
                                    bs2, ba2)
        else:
            st = _tc_final(st, s_pad, t_pad, ea, linked_state_W, bs2)
    return st[None]
